# baseline (speedup 1.0000x reference)
import math

import jax
import jax.numpy as jnp
from jax import lax
from jax.experimental import pallas as pl
from jax.experimental.pallas import tpu as pltpu
from jax.experimental.pallas import tpu_sc as plsc

EMB = 128
ENTS = 10000
NRELS = 400
NEDGE = 320000
NC = 2
NS = 16
L = 16
NW = NC * NS
EPT = NEDGE // NW
KE = 80
NCHUNK = EPT // KE
CPB = 25
BLK = CPB * KE
NZCH = ENTS // KE
ZIT = -(-NZCH // NS)
CIT = -(-NZCH // NW)
BATCH = 1024
SEQ = 10
SEQTOT = BATCH * SEQ
SPW = SEQTOT // NW
KS = 80
NSC = SPW // KS

_MESH = dict(core_axis_name="c", subcore_axis_name="s")


def _edge_body(src_hbm, dst_hbm, typ_hbm, ent_hbm, rel_hbm,
               agg0, agg1, deg0, deg1, agg_sh, deg_sh,
               src_big, typ_big, dst_v0, dst_v1,
               ent_b0, ent_b1, rel_b0, rel_b1, ones_b, zdeg,
               gsem0, gsem1, ssem0, ssem1):
    c = lax.axis_index("c")
    s = lax.axis_index("s")
    wid = c * NS + s
    ebase = wid * EPT
    zv = jnp.zeros((L,), jnp.float32)
    ov = jnp.ones((L,), jnp.float32)
    ENT = (ent_b0, ent_b1)
    REL = (rel_b0, rel_b1)
    DSTV = (dst_v0, dst_v1)
    GS = (gsem0, gsem1)
    SS = (ssem0, ssem1)

    def _zfill(r, _):
        for jj in range(EMB // L):
            ent_b0[r, pl.ds(jj * L, L)] = zv
        return 0
    lax.fori_loop(0, KE, _zfill, 0)

    def _zfill1(g, _):
        zdeg[pl.ds(g * L, L)] = zv
        ones_b[pl.ds(g * L, L)] = ov
        return 0
    lax.fori_loop(0, KE // L, _zfill1, 0)

    def _zchunk(i, _):
        j = i * NS + s

        @pl.when(j < NZCH)
        def _():
            pltpu.sync_copy(ent_b0, agg_sh.at[pl.ds(j * KE, KE)])
            pltpu.sync_copy(zdeg, deg_sh.at[pl.ds(j * KE, KE)])
        return 0
    lax.fori_loop(0, ZIT, _zchunk, 0)
    plsc.subcore_barrier()

    def load_block(bidx):
        b0 = ebase + bidx * BLK
        pltpu.sync_copy(src_hbm.at[pl.ds(b0, BLK)], src_big)
        pltpu.sync_copy(typ_hbm.at[pl.ds(b0, BLK)], typ_big)

    def issue(cn, q):
        off = (cn % CPB) * KE
        pltpu.async_copy(dst_hbm.at[pl.ds(ebase + cn * KE, KE)], DSTV[q],
                         GS[q])
        pltpu.async_copy(ent_hbm.at[src_big.at[pl.ds(off, KE)]], ENT[q],
                         GS[q])
        pltpu.async_copy(rel_hbm.at[typ_big.at[pl.ds(off, KE)]], REL[q],
                         GS[q])

    def drain_g(p):
        pltpu.make_async_copy(dst_hbm.at[pl.ds(0, KE)], DSTV[p],
                              GS[p]).wait()
        pltpu.make_async_copy(agg0.at[pl.ds(0, KE)], ENT[p], GS[p]).wait()
        pltpu.make_async_copy(agg0.at[pl.ds(0, KE)], REL[p], GS[p]).wait()

    def drain_s(p):
        pltpu.make_async_copy(agg0.at[pl.ds(0, KE)], ENT[p], SS[p]).wait()
        pltpu.make_async_copy(deg0.at[pl.ds(0, KE)], ones_b, SS[p]).wait()

    def mult(p):
        @plsc.parallel_loop(0, KE, unroll=8)
        def _(r):
            for jj in range(EMB // L):
                sl = pl.ds(jj * L, L)
                ENT[p][r, sl] = ENT[p][r, sl] * REL[p][r, sl]

    def scat(p):
        pltpu.async_copy(ENT[p], agg_sh.at[DSTV[p]], SS[p], add=True)
        pltpu.async_copy(ones_b, deg_sh.at[DSTV[p]], SS[p], add=True)

    def step(cn, p, do_next):
        q = 1 - p
        drain_g(p)

        @pl.when(cn >= 1)
        def _():
            drain_s(q)
        if do_next:
            nxt = cn + 1

            @pl.when(nxt % CPB == 0)
            def _():
                load_block(nxt // CPB)
            issue(nxt, q)
        mult(p)
        scat(p)

    load_block(0)
    issue(0, 0)

    def pair(i, _):
        step(2 * i, 0, True)
        step(2 * i + 1, 1, True)
        return 0
    lax.fori_loop(0, (NCHUNK - 1) // 2, pair, 0)
    step(NCHUNK - 1, 0, False)
    drain_s(0)
    plsc.subcore_barrier()

    def _wout(agg_out, deg_out):
        def wchunk(i, _):
            j = i * NS + s

            @pl.when(j < NZCH)
            def _():
                r0 = j * KE
                pltpu.sync_copy(agg_sh.at[pl.ds(r0, KE)], rel_b0)
                pltpu.sync_copy(rel_b0, agg_out.at[pl.ds(r0, KE)])
                pltpu.sync_copy(deg_sh.at[pl.ds(r0, KE)], zdeg)
                pltpu.sync_copy(zdeg, deg_out.at[pl.ds(r0, KE)])
            return 0
        lax.fori_loop(0, ZIT, wchunk, 0)

    @pl.when(c == 0)
    def _():
        _wout(agg0, deg0)

    @pl.when(c == 1)
    def _():
        _wout(agg1, deg1)


def _comb_body(ent_hbm, agg0, agg1, deg0, deg1, e_hbm,
               a0_b, a1_b, d0_b, d1_b, ent_b):
    c = lax.axis_index("c")
    s = lax.axis_index("s")
    wid = c * NS + s

    def chunk(i, _):
        j = i * NW + wid

        @pl.when(j < NZCH)
        def _():
            r0 = j * KE
            pltpu.sync_copy(ent_hbm.at[pl.ds(r0, KE)], ent_b)
            pltpu.sync_copy(agg0.at[pl.ds(r0, KE)], a0_b)
            pltpu.sync_copy(agg1.at[pl.ds(r0, KE)], a1_b)
            pltpu.sync_copy(deg0.at[pl.ds(r0, KE)], d0_b)
            pltpu.sync_copy(deg1.at[pl.ds(r0, KE)], d1_b)

            def pgrp(g, _):
                d16 = jnp.maximum(
                    d0_b[pl.ds(g * L, L)] + d1_b[pl.ds(g * L, L)], 1.0)
                for rr in range(L):
                    r = g * L + rr
                    dv = d16[rr]
                    for jj in range(EMB // L):
                        sl = pl.ds(jj * L, L)
                        ent_b[r, sl] = ent_b[r, sl] + jnp.maximum(
                            (a0_b[r, sl] + a1_b[r, sl]) / dv, 0.0)
                return 0
            lax.fori_loop(0, KE // L, pgrp, 0)
            pltpu.sync_copy(ent_b, e_hbm.at[pl.ds(r0, KE)])
        return 0
    lax.fori_loop(0, CIT, chunk, 0)


def _gather_body(seq_hbm, tid_hbm, e_hbm, te_hbm, pair_hbm,
                 sid_v, tid_v, e_b, t_b, sem0, sem1):
    c = lax.axis_index("c")
    s = lax.axis_index("s")
    wid = c * NS + s

    def chunk(it, _):
        base = wid * SPW + it * KS
        pltpu.sync_copy(seq_hbm.at[pl.ds(base, KS)], sid_v)
        pltpu.sync_copy(tid_hbm.at[pl.ds(base, KS)], tid_v)
        cp0 = pltpu.async_copy(e_hbm.at[sid_v], e_b, sem0)
        cp1 = pltpu.async_copy(te_hbm.at[tid_v], t_b, sem1)
        cp0.wait()
        cp1.wait()

        def row(r, _):
            for jj in range(EMB // L):
                sl = pl.ds(jj * L, L)
                e_b[r, sl] = e_b[r, sl] + t_b[r, sl]
            return 0
        lax.fori_loop(0, KS, row, 0)
        pltpu.sync_copy(e_b, pair_hbm.at[pl.ds(base, KS)])
        return 0
    lax.fori_loop(0, NSC, chunk, 0)


def _decode_body(pair_ref, e_ref, w_ref, b_ref, out_ref, rep_ref):
    t = pl.program_id(0)

    @pl.when(t == 0)
    def _():
        acc = jnp.zeros((BATCH, EMB), jnp.float32)
        for p in range(SEQ):
            x = pair_ref[p]
            u = jnp.mean(x, axis=-1, keepdims=True)
            xc = x - u
            s2 = jnp.mean(xc * xc, axis=-1, keepdims=True)
            xn = xc / jnp.sqrt(s2 + 1e-12)
            acc = acc + (xn * w_ref[...] + b_ref[...])
        scale = 1.0 / (SEQ * math.sqrt(float(EMB)))
        rep_ref[...] = acc * scale

    out_ref[...] = lax.dot_general(
        rep_ref[...], e_ref[...], (((1,), (1,)), ((), ())),
        preferred_element_type=jnp.float32,
        precision=lax.Precision.HIGHEST)


def _edge_call(src, dst, typ, emb_ent, emb_rel):
    f32 = jnp.float32
    return pl.kernel(
        _edge_body,
        out_type=[jax.ShapeDtypeStruct((ENTS, EMB), f32),
                  jax.ShapeDtypeStruct((ENTS, EMB), f32),
                  jax.ShapeDtypeStruct((ENTS,), f32),
                  jax.ShapeDtypeStruct((ENTS,), f32)],
        mesh=plsc.VectorSubcoreMesh(**_MESH),
        scratch_types=[
            pltpu.VMEM_SHARED((ENTS, EMB), f32),
            pltpu.VMEM_SHARED((ENTS,), f32),
            pltpu.VMEM((BLK,), jnp.int32),
            pltpu.VMEM((BLK,), jnp.int32),
            pltpu.VMEM((KE,), jnp.int32),
            pltpu.VMEM((KE,), jnp.int32),
            pltpu.VMEM((KE, EMB), f32),
            pltpu.VMEM((KE, EMB), f32),
            pltpu.VMEM((KE, EMB), f32),
            pltpu.VMEM((KE, EMB), f32),
            pltpu.VMEM((KE,), f32),
            pltpu.VMEM((KE,), f32),
            pltpu.SemaphoreType.DMA,
            pltpu.SemaphoreType.DMA,
            pltpu.SemaphoreType.DMA,
            pltpu.SemaphoreType.DMA,
        ],
    )(src, dst, typ, emb_ent, emb_rel)


def _comb_call(emb_ent, agg0, agg1, deg0, deg1):
    f32 = jnp.float32
    return pl.kernel(
        _comb_body,
        out_type=jax.ShapeDtypeStruct((ENTS, EMB), f32),
        mesh=plsc.VectorSubcoreMesh(**_MESH),
        scratch_types=[
            pltpu.VMEM((KE, EMB), f32),
            pltpu.VMEM((KE, EMB), f32),
            pltpu.VMEM((KE,), f32),
            pltpu.VMEM((KE,), f32),
            pltpu.VMEM((KE, EMB), f32),
        ],
    )(emb_ent, agg0, agg1, deg0, deg1)


def _gather_call(seq_t, tid_t, e_embs, time_emb):
    f32 = jnp.float32
    return pl.kernel(
        _gather_body,
        out_type=jax.ShapeDtypeStruct((SEQTOT, EMB), f32),
        mesh=plsc.VectorSubcoreMesh(**_MESH),
        scratch_types=[
            pltpu.VMEM((KS,), jnp.int32),
            pltpu.VMEM((KS,), jnp.int32),
            pltpu.VMEM((KS, EMB), f32),
            pltpu.VMEM((KS, EMB), f32),
            pltpu.SemaphoreType.DMA,
            pltpu.SemaphoreType.DMA,
        ],
    )(seq_t, tid_t, e_embs, time_emb)


def _decode_call(pair3, e_embs, w2, b2):
    nblk = 10
    return pl.pallas_call(
        _decode_body,
        grid=(nblk,),
        in_specs=[
            pl.BlockSpec((SEQ, BATCH, EMB), lambda t: (0, 0, 0)),
            pl.BlockSpec((1024, EMB), lambda t: (t, 0)),
            pl.BlockSpec((1, EMB), lambda t: (0, 0)),
            pl.BlockSpec((1, EMB), lambda t: (0, 0)),
        ],
        out_specs=pl.BlockSpec((BATCH, 1024), lambda t: (0, t)),
        out_shape=jax.ShapeDtypeStruct((BATCH, ENTS), jnp.float32),
        scratch_shapes=[pltpu.VMEM((BATCH, EMB), jnp.float32)],
    )(pair3, e_embs, w2, b2)


def kernel(sequence, time_ids, edge_index, edge_type, emb_ent, emb_rel,
           time_emb, ln_weight, ln_bias):
    i32 = jnp.int32
    src = edge_index[0].astype(i32)
    dst = edge_index[1].astype(i32)
    typ = edge_type.astype(i32) + (
        jnp.arange(NEDGE, dtype=i32) // EPT) * NRELS
    seq_t = sequence.T.reshape(-1).astype(i32)
    tid_t = time_ids.T.reshape(-1).astype(i32)
    rel_rep = jnp.tile(emb_rel, (NW, 1))
    agg0, agg1, deg0, deg1 = _edge_call(src, dst, typ, emb_ent, rel_rep)
    e_embs = _comb_call(emb_ent, agg0, agg1, deg0, deg1)
    pair = _gather_call(seq_t, tid_t, e_embs, time_emb)
    pair3 = pair.reshape(SEQ, BATCH, EMB)
    return _decode_call(pair3, e_embs,
                        ln_weight.reshape(1, EMB).astype(jnp.float32),
                        ln_bias.reshape(1, EMB).astype(jnp.float32))


# speedup vs baseline: 1.0297x; 1.0297x over previous
import math

import jax
import jax.numpy as jnp
from jax import lax
from jax.experimental import pallas as pl
from jax.experimental.pallas import tpu as pltpu
from jax.experimental.pallas import tpu_sc as plsc

EMB = 128
ENTS = 10000
NRELS = 400
NEDGE = 320000
NC = 2
NS = 16
L = 16
NW = NC * NS
EPT = NEDGE // NW
KE = 80
NCHUNK = EPT // KE
CPB = 25
BLK = CPB * KE
NZCH = ENTS // KE
ZIT = -(-NZCH // NS)
CIT = -(-NZCH // NW)
BATCH = 1024
SEQ = 10
SEQTOT = BATCH * SEQ
SPW = SEQTOT // NW
KS = 80
NSC = SPW // KS

_MESH = dict(core_axis_name="c", subcore_axis_name="s")


def _edge_body(src_hbm, dst_hbm, typ_hbm, ent_hbm, rel_hbm,
               agg0, agg1, deg0, deg1, agg_sh, deg_sh,
               src_big, typ_big, dst_v0, dst_v1,
               ent_b0, ent_b1, rel_b0, rel_b1, ones_b, zdeg,
               gsem0, gsem1, ssem0, ssem1):
    c = lax.axis_index("c")
    s = lax.axis_index("s")
    wid = c * NS + s
    ebase = wid * EPT
    zv = jnp.zeros((L,), jnp.float32)
    ov = jnp.ones((L,), jnp.float32)
    ENT = (ent_b0, ent_b1)
    REL = (rel_b0, rel_b1)
    DSTV = (dst_v0, dst_v1)
    GS = (gsem0, gsem1)
    SS = (ssem0, ssem1)

    def _zfill(r, _):
        for jj in range(EMB // L):
            ent_b0[r, pl.ds(jj * L, L)] = zv
        return 0
    lax.fori_loop(0, KE, _zfill, 0)

    def _zfill1(g, _):
        zdeg[pl.ds(g * L, L)] = zv
        ones_b[pl.ds(g * L, L)] = ov
        return 0
    lax.fori_loop(0, KE // L, _zfill1, 0)

    def _zchunk(i, _):
        j = i * NS + s

        @pl.when(j < NZCH)
        def _():
            pltpu.sync_copy(ent_b0, agg_sh.at[pl.ds(j * KE, KE)])
            pltpu.sync_copy(zdeg, deg_sh.at[pl.ds(j * KE, KE)])
        return 0
    lax.fori_loop(0, ZIT, _zchunk, 0)
    plsc.subcore_barrier()

    def load_block(bidx):
        b0 = ebase + bidx * BLK
        pltpu.sync_copy(src_hbm.at[pl.ds(b0, BLK)], src_big)
        pltpu.sync_copy(typ_hbm.at[pl.ds(b0, BLK)], typ_big)

    def issue(cn, q):
        off = (cn % CPB) * KE
        pltpu.async_copy(dst_hbm.at[pl.ds(ebase + cn * KE, KE)], DSTV[q],
                         GS[q])
        pltpu.async_copy(ent_hbm.at[src_big.at[pl.ds(off, KE)]], ENT[q],
                         GS[q])
        pltpu.async_copy(rel_hbm.at[typ_big.at[pl.ds(off, KE)]], REL[q],
                         GS[q])

    def drain_g(p):
        pltpu.make_async_copy(dst_hbm.at[pl.ds(0, KE)], DSTV[p],
                              GS[p]).wait()
        pltpu.make_async_copy(agg0.at[pl.ds(0, KE)], ENT[p], GS[p]).wait()
        pltpu.make_async_copy(agg0.at[pl.ds(0, KE)], REL[p], GS[p]).wait()

    def drain_s(p):
        pltpu.make_async_copy(agg0.at[pl.ds(0, KE)], ENT[p], SS[p]).wait()
        pltpu.make_async_copy(deg0.at[pl.ds(0, KE)], ones_b, SS[p]).wait()

    def mult(p):
        @plsc.parallel_loop(0, KE, unroll=16)
        def _(r):
            for jj in range(EMB // L):
                sl = pl.ds(jj * L, L)
                ENT[p][r, sl] = ENT[p][r, sl] * REL[p][r, sl]

    def scat(p):
        pltpu.async_copy(ENT[p], agg_sh.at[DSTV[p]], SS[p], add=True)
        pltpu.async_copy(ones_b, deg_sh.at[DSTV[p]], SS[p], add=True)

    def step(cn, p, do_next):
        q = 1 - p
        drain_g(p)

        @pl.when(cn >= 1)
        def _():
            drain_s(q)
        if do_next:
            nxt = cn + 1

            @pl.when(nxt % CPB == 0)
            def _():
                load_block(nxt // CPB)
            issue(nxt, q)
        mult(p)
        scat(p)

    load_block(0)
    issue(0, 0)

    def pair(i, _):
        step(2 * i, 0, True)
        step(2 * i + 1, 1, True)
        return 0
    lax.fori_loop(0, (NCHUNK - 1) // 2, pair, 0)
    step(NCHUNK - 1, 0, False)
    drain_s(0)
    plsc.subcore_barrier()

    def _wout(agg_out, deg_out):
        def wchunk(i, _):
            j = i * NS + s

            @pl.when(j < NZCH)
            def _():
                r0 = j * KE
                pltpu.sync_copy(agg_sh.at[pl.ds(r0, KE)], rel_b0)
                pltpu.sync_copy(rel_b0, agg_out.at[pl.ds(r0, KE)])
                pltpu.sync_copy(deg_sh.at[pl.ds(r0, KE)], zdeg)
                pltpu.sync_copy(zdeg, deg_out.at[pl.ds(r0, KE)])
            return 0
        lax.fori_loop(0, ZIT, wchunk, 0)

    @pl.when(c == 0)
    def _():
        _wout(agg0, deg0)

    @pl.when(c == 1)
    def _():
        _wout(agg1, deg1)


def _comb_body(ent_hbm, agg0, agg1, deg0, deg1, e_hbm,
               a0_b, a1_b, d0_b, d1_b, ent_b):
    c = lax.axis_index("c")
    s = lax.axis_index("s")
    wid = c * NS + s

    def chunk(i, _):
        j = i * NW + wid

        @pl.when(j < NZCH)
        def _():
            r0 = j * KE
            pltpu.sync_copy(ent_hbm.at[pl.ds(r0, KE)], ent_b)
            pltpu.sync_copy(agg0.at[pl.ds(r0, KE)], a0_b)
            pltpu.sync_copy(agg1.at[pl.ds(r0, KE)], a1_b)
            pltpu.sync_copy(deg0.at[pl.ds(r0, KE)], d0_b)
            pltpu.sync_copy(deg1.at[pl.ds(r0, KE)], d1_b)

            def pgrp(g, _):
                d16 = jnp.maximum(
                    d0_b[pl.ds(g * L, L)] + d1_b[pl.ds(g * L, L)], 1.0)
                for rr in range(L):
                    r = g * L + rr
                    dv = d16[rr]
                    for jj in range(EMB // L):
                        sl = pl.ds(jj * L, L)
                        ent_b[r, sl] = ent_b[r, sl] + jnp.maximum(
                            (a0_b[r, sl] + a1_b[r, sl]) / dv, 0.0)
                return 0
            lax.fori_loop(0, KE // L, pgrp, 0)
            pltpu.sync_copy(ent_b, e_hbm.at[pl.ds(r0, KE)])
        return 0
    lax.fori_loop(0, CIT, chunk, 0)


def _gather_body(seq_hbm, tid_hbm, e_hbm, te_hbm, pair_hbm,
                 sid_v, tid_v, e_b, t_b, sem0, sem1):
    c = lax.axis_index("c")
    s = lax.axis_index("s")
    wid = c * NS + s

    def chunk(it, _):
        base = wid * SPW + it * KS
        pltpu.sync_copy(seq_hbm.at[pl.ds(base, KS)], sid_v)
        pltpu.sync_copy(tid_hbm.at[pl.ds(base, KS)], tid_v)
        cp0 = pltpu.async_copy(e_hbm.at[sid_v], e_b, sem0)
        cp1 = pltpu.async_copy(te_hbm.at[tid_v], t_b, sem1)
        cp0.wait()
        cp1.wait()

        def row(r, _):
            for jj in range(EMB // L):
                sl = pl.ds(jj * L, L)
                e_b[r, sl] = e_b[r, sl] + t_b[r, sl]
            return 0
        lax.fori_loop(0, KS, row, 0)
        pltpu.sync_copy(e_b, pair_hbm.at[pl.ds(base, KS)])
        return 0
    lax.fori_loop(0, NSC, chunk, 0)


def _decode_body(pair_ref, e_ref, w_ref, b_ref, out_ref, rep_ref):
    t = pl.program_id(0)

    @pl.when(t == 0)
    def _():
        acc = jnp.zeros((BATCH, EMB), jnp.float32)
        for p in range(SEQ):
            x = pair_ref[p]
            u = jnp.mean(x, axis=-1, keepdims=True)
            xc = x - u
            s2 = jnp.mean(xc * xc, axis=-1, keepdims=True)
            xn = xc / jnp.sqrt(s2 + 1e-12)
            acc = acc + (xn * w_ref[...] + b_ref[...])
        scale = 1.0 / (SEQ * math.sqrt(float(EMB)))
        rep_ref[...] = acc * scale

    out_ref[...] = lax.dot_general(
        rep_ref[...], e_ref[...], (((1,), (1,)), ((), ())),
        preferred_element_type=jnp.float32,
        precision=lax.Precision.HIGHEST)


def _edge_call(src, dst, typ, emb_ent, emb_rel):
    f32 = jnp.float32
    return pl.kernel(
        _edge_body,
        out_type=[jax.ShapeDtypeStruct((ENTS, EMB), f32),
                  jax.ShapeDtypeStruct((ENTS, EMB), f32),
                  jax.ShapeDtypeStruct((ENTS,), f32),
                  jax.ShapeDtypeStruct((ENTS,), f32)],
        mesh=plsc.VectorSubcoreMesh(**_MESH),
        scratch_types=[
            pltpu.VMEM_SHARED((ENTS, EMB), f32),
            pltpu.VMEM_SHARED((ENTS,), f32),
            pltpu.VMEM((BLK,), jnp.int32),
            pltpu.VMEM((BLK,), jnp.int32),
            pltpu.VMEM((KE,), jnp.int32),
            pltpu.VMEM((KE,), jnp.int32),
            pltpu.VMEM((KE, EMB), f32),
            pltpu.VMEM((KE, EMB), f32),
            pltpu.VMEM((KE, EMB), f32),
            pltpu.VMEM((KE, EMB), f32),
            pltpu.VMEM((KE,), f32),
            pltpu.VMEM((KE,), f32),
            pltpu.SemaphoreType.DMA,
            pltpu.SemaphoreType.DMA,
            pltpu.SemaphoreType.DMA,
            pltpu.SemaphoreType.DMA,
        ],
    )(src, dst, typ, emb_ent, emb_rel)


def _comb_call(emb_ent, agg0, agg1, deg0, deg1):
    f32 = jnp.float32
    return pl.kernel(
        _comb_body,
        out_type=jax.ShapeDtypeStruct((ENTS, EMB), f32),
        mesh=plsc.VectorSubcoreMesh(**_MESH),
        scratch_types=[
            pltpu.VMEM((KE, EMB), f32),
            pltpu.VMEM((KE, EMB), f32),
            pltpu.VMEM((KE,), f32),
            pltpu.VMEM((KE,), f32),
            pltpu.VMEM((KE, EMB), f32),
        ],
    )(emb_ent, agg0, agg1, deg0, deg1)


def _gather_call(seq_t, tid_t, e_embs, time_emb):
    f32 = jnp.float32
    return pl.kernel(
        _gather_body,
        out_type=jax.ShapeDtypeStruct((SEQTOT, EMB), f32),
        mesh=plsc.VectorSubcoreMesh(**_MESH),
        scratch_types=[
            pltpu.VMEM((KS,), jnp.int32),
            pltpu.VMEM((KS,), jnp.int32),
            pltpu.VMEM((KS, EMB), f32),
            pltpu.VMEM((KS, EMB), f32),
            pltpu.SemaphoreType.DMA,
            pltpu.SemaphoreType.DMA,
        ],
    )(seq_t, tid_t, e_embs, time_emb)


def _decode_call(pair3, e_embs, w2, b2):
    nblk = 10
    return pl.pallas_call(
        _decode_body,
        grid=(nblk,),
        in_specs=[
            pl.BlockSpec((SEQ, BATCH, EMB), lambda t: (0, 0, 0)),
            pl.BlockSpec((1024, EMB), lambda t: (t, 0)),
            pl.BlockSpec((1, EMB), lambda t: (0, 0)),
            pl.BlockSpec((1, EMB), lambda t: (0, 0)),
        ],
        out_specs=pl.BlockSpec((BATCH, 1024), lambda t: (0, t)),
        out_shape=jax.ShapeDtypeStruct((BATCH, ENTS), jnp.float32),
        scratch_shapes=[pltpu.VMEM((BATCH, EMB), jnp.float32)],
    )(pair3, e_embs, w2, b2)


def kernel(sequence, time_ids, edge_index, edge_type, emb_ent, emb_rel,
           time_emb, ln_weight, ln_bias):
    i32 = jnp.int32
    src = edge_index[0].astype(i32)
    dst = edge_index[1].astype(i32)
    typ = edge_type.astype(i32) + (
        jnp.arange(NEDGE, dtype=i32) // EPT) * NRELS
    seq_t = sequence.T.reshape(-1).astype(i32)
    tid_t = time_ids.T.reshape(-1).astype(i32)
    rel_rep = jnp.tile(emb_rel, (NW, 1))
    agg0, agg1, deg0, deg1 = _edge_call(src, dst, typ, emb_ent, rel_rep)
    e_embs = _comb_call(emb_ent, agg0, agg1, deg0, deg1)
    pair = _gather_call(seq_t, tid_t, e_embs, time_emb)
    pair3 = pair.reshape(SEQ, BATCH, EMB)
    return _decode_call(pair3, e_embs,
                        ln_weight.reshape(1, EMB).astype(jnp.float32),
                        ln_bias.reshape(1, EMB).astype(jnp.float32))


# speedup vs baseline: 1.0547x; 1.0243x over previous
import math

import jax
import jax.numpy as jnp
from jax import lax
from jax.experimental import pallas as pl
from jax.experimental.pallas import tpu as pltpu
from jax.experimental.pallas import tpu_sc as plsc

EMB = 128
ENTS = 10000
NRELS = 400
NEDGE = 320000
NC = 2
NS = 16
L = 16
NW = NC * NS
EPT = NEDGE // NW
KE = 80
NCHUNK = EPT // KE
CPB = 25
BLK = CPB * KE
NZCH = ENTS // KE
ZIT = -(-NZCH // NS)
CIT = -(-NZCH // NW)
BATCH = 1024
SEQ = 10
SEQTOT = BATCH * SEQ
SPW = SEQTOT // NW
KS = 80
NSC = SPW // KS

_MESH = dict(core_axis_name="c", subcore_axis_name="s")


def _edge_body(src_hbm, dst_hbm, typ_hbm, ent_hbm, rel_hbm,
               agg0, agg1, deg0, deg1, agg_sh, deg_sh,
               src_big, typ_big, dst_v0, dst_v1,
               ent_b0, ent_b1, rel_b0, rel_b1, ones_b, zdeg,
               gsem0, gsem1, ssem0, ssem1):
    c = lax.axis_index("c")
    s = lax.axis_index("s")
    wid = c * NS + s
    ebase = wid * EPT
    zv = jnp.zeros((L,), jnp.float32)
    ov = jnp.ones((L,), jnp.float32)
    ENT = (ent_b0, ent_b1)
    REL = (rel_b0, rel_b1)
    DSTV = (dst_v0, dst_v1)
    GS = (gsem0, gsem1)
    SS = (ssem0, ssem1)

    def _zfill(r, _):
        for jj in range(EMB // L):
            ent_b0[r, pl.ds(jj * L, L)] = zv
        return 0
    lax.fori_loop(0, KE, _zfill, 0)

    def _zfill1(g, _):
        zdeg[pl.ds(g * L, L)] = zv
        ones_b[pl.ds(g * L, L)] = ov
        return 0
    lax.fori_loop(0, KE // L, _zfill1, 0)

    def _zchunk(i, _):
        j = i * NS + s

        @pl.when(j < NZCH)
        def _():
            pltpu.sync_copy(ent_b0, agg_sh.at[pl.ds(j * KE, KE)])
            pltpu.sync_copy(zdeg, deg_sh.at[pl.ds(j * KE, KE)])
        return 0
    lax.fori_loop(0, ZIT, _zchunk, 0)
    plsc.subcore_barrier()

    def load_block(bidx):
        b0 = ebase + bidx * BLK
        pltpu.sync_copy(src_hbm.at[pl.ds(b0, BLK)], src_big)
        pltpu.sync_copy(typ_hbm.at[pl.ds(b0, BLK)], typ_big)

    def issue(cn, q):
        off = (cn % CPB) * KE
        pltpu.async_copy(dst_hbm.at[pl.ds(ebase + cn * KE, KE)], DSTV[q],
                         GS[q])
        pltpu.async_copy(ent_hbm.at[src_big.at[pl.ds(off, KE)]], ENT[q],
                         GS[q])
        pltpu.async_copy(rel_hbm.at[typ_big.at[pl.ds(off, KE)]], REL[q],
                         GS[q])

    def drain_g(p):
        pltpu.make_async_copy(dst_hbm.at[pl.ds(0, KE)], DSTV[p],
                              GS[p]).wait()
        pltpu.make_async_copy(agg0.at[pl.ds(0, KE)], ENT[p], GS[p]).wait()
        pltpu.make_async_copy(agg0.at[pl.ds(0, KE)], REL[p], GS[p]).wait()

    def drain_s(p):
        pltpu.make_async_copy(agg0.at[pl.ds(0, KE)], ENT[p], SS[p]).wait()
        pltpu.make_async_copy(deg0.at[pl.ds(0, KE)], ones_b, SS[p]).wait()

    def mult(p):
        @plsc.parallel_loop(0, KE, unroll=16)
        def _(r):
            for jj in range(EMB // L):
                sl = pl.ds(jj * L, L)
                ENT[p][r, sl] = ENT[p][r, sl] * REL[p][r, sl]

    def scat(p):
        pltpu.async_copy(ENT[p], agg_sh.at[DSTV[p]], SS[p], add=True)
        pltpu.async_copy(ones_b, deg_sh.at[DSTV[p]], SS[p], add=True)

    def step(cn, p, do_next):
        q = 1 - p
        drain_g(p)

        @pl.when(cn >= 1)
        def _():
            drain_s(q)
        if do_next:
            nxt = cn + 1

            @pl.when(nxt % CPB == 0)
            def _():
                load_block(nxt // CPB)
            issue(nxt, q)
        mult(p)
        scat(p)

    load_block(0)
    issue(0, 0)

    def pair(i, _):
        step(2 * i, 0, True)
        step(2 * i + 1, 1, True)
        return 0
    lax.fori_loop(0, (NCHUNK - 1) // 2, pair, 0)
    step(NCHUNK - 1, 0, False)
    drain_s(0)
    plsc.subcore_barrier()

    def _wout(agg_out, deg_out):
        def wchunk(i, _):
            j = i * NS + s

            @pl.when(j < NZCH)
            def _():
                r0 = j * KE
                pltpu.sync_copy(agg_sh.at[pl.ds(r0, KE)], rel_b0)
                pltpu.sync_copy(rel_b0, agg_out.at[pl.ds(r0, KE)])
                pltpu.sync_copy(deg_sh.at[pl.ds(r0, KE)], zdeg)
                pltpu.sync_copy(zdeg, deg_out.at[pl.ds(r0, KE)])
            return 0
        lax.fori_loop(0, ZIT, wchunk, 0)

    @pl.when(c == 0)
    def _():
        _wout(agg0, deg0)

    @pl.when(c == 1)
    def _():
        _wout(agg1, deg1)


def _comb_body(ent_hbm, agg0, agg1, deg0, deg1, e_hbm,
               a0_b, a1_b, d0_b, d1_b, ent_b, csem):
    c = lax.axis_index("c")
    s = lax.axis_index("s")
    wid = c * NS + s

    def chunk(i, _):
        j = i * NW + wid

        @pl.when(j < NZCH)
        def _():
            r0 = j * KE
            cps = [
                pltpu.async_copy(ent_hbm.at[pl.ds(r0, KE)], ent_b, csem),
                pltpu.async_copy(agg0.at[pl.ds(r0, KE)], a0_b, csem),
                pltpu.async_copy(agg1.at[pl.ds(r0, KE)], a1_b, csem),
                pltpu.async_copy(deg0.at[pl.ds(r0, KE)], d0_b, csem),
                pltpu.async_copy(deg1.at[pl.ds(r0, KE)], d1_b, csem),
            ]
            for cp in cps:
                cp.wait()

            def pgrp(g, _):
                d16 = jnp.maximum(
                    d0_b[pl.ds(g * L, L)] + d1_b[pl.ds(g * L, L)], 1.0)
                for rr in range(L):
                    r = g * L + rr
                    dv = d16[rr]
                    for jj in range(EMB // L):
                        sl = pl.ds(jj * L, L)
                        ent_b[r, sl] = ent_b[r, sl] + jnp.maximum(
                            (a0_b[r, sl] + a1_b[r, sl]) / dv, 0.0)
                return 0
            lax.fori_loop(0, KE // L, pgrp, 0)
            pltpu.sync_copy(ent_b, e_hbm.at[pl.ds(r0, KE)])
        return 0
    lax.fori_loop(0, CIT, chunk, 0)


def _gather_body(seq_hbm, tid_hbm, e_hbm, te_hbm, pair_hbm,
                 sid_v, tid_v, e_b, t_b, sem0, sem1):
    c = lax.axis_index("c")
    s = lax.axis_index("s")
    wid = c * NS + s

    def chunk(it, _):
        base = wid * SPW + it * KS
        pltpu.sync_copy(seq_hbm.at[pl.ds(base, KS)], sid_v)
        pltpu.sync_copy(tid_hbm.at[pl.ds(base, KS)], tid_v)
        cp0 = pltpu.async_copy(e_hbm.at[sid_v], e_b, sem0)
        cp1 = pltpu.async_copy(te_hbm.at[tid_v], t_b, sem1)
        cp0.wait()
        cp1.wait()

        def row(r, _):
            for jj in range(EMB // L):
                sl = pl.ds(jj * L, L)
                e_b[r, sl] = e_b[r, sl] + t_b[r, sl]
            return 0
        lax.fori_loop(0, KS, row, 0)
        pltpu.sync_copy(e_b, pair_hbm.at[pl.ds(base, KS)])
        return 0
    lax.fori_loop(0, NSC, chunk, 0)


def _decode_body(pair_ref, e_ref, w_ref, b_ref, out_ref, rep_ref):
    t = pl.program_id(0)

    @pl.when(t == 0)
    def _():
        acc = jnp.zeros((BATCH, EMB), jnp.float32)
        for p in range(SEQ):
            x = pair_ref[p]
            u = jnp.mean(x, axis=-1, keepdims=True)
            xc = x - u
            s2 = jnp.mean(xc * xc, axis=-1, keepdims=True)
            xn = xc / jnp.sqrt(s2 + 1e-12)
            acc = acc + (xn * w_ref[...] + b_ref[...])
        scale = 1.0 / (SEQ * math.sqrt(float(EMB)))
        rep_ref[...] = acc * scale

    out_ref[...] = lax.dot_general(
        rep_ref[...], e_ref[...], (((1,), (1,)), ((), ())),
        preferred_element_type=jnp.float32,
        precision=lax.Precision.HIGHEST)


def _edge_call(src, dst, typ, emb_ent, emb_rel):
    f32 = jnp.float32
    return pl.kernel(
        _edge_body,
        out_type=[jax.ShapeDtypeStruct((ENTS, EMB), f32),
                  jax.ShapeDtypeStruct((ENTS, EMB), f32),
                  jax.ShapeDtypeStruct((ENTS,), f32),
                  jax.ShapeDtypeStruct((ENTS,), f32)],
        mesh=plsc.VectorSubcoreMesh(**_MESH),
        scratch_types=[
            pltpu.VMEM_SHARED((ENTS, EMB), f32),
            pltpu.VMEM_SHARED((ENTS,), f32),
            pltpu.VMEM((BLK,), jnp.int32),
            pltpu.VMEM((BLK,), jnp.int32),
            pltpu.VMEM((KE,), jnp.int32),
            pltpu.VMEM((KE,), jnp.int32),
            pltpu.VMEM((KE, EMB), f32),
            pltpu.VMEM((KE, EMB), f32),
            pltpu.VMEM((KE, EMB), f32),
            pltpu.VMEM((KE, EMB), f32),
            pltpu.VMEM((KE,), f32),
            pltpu.VMEM((KE,), f32),
            pltpu.SemaphoreType.DMA,
            pltpu.SemaphoreType.DMA,
            pltpu.SemaphoreType.DMA,
            pltpu.SemaphoreType.DMA,
        ],
    )(src, dst, typ, emb_ent, emb_rel)


def _comb_call(emb_ent, agg0, agg1, deg0, deg1):
    f32 = jnp.float32
    return pl.kernel(
        _comb_body,
        out_type=jax.ShapeDtypeStruct((ENTS, EMB), f32),
        mesh=plsc.VectorSubcoreMesh(**_MESH),
        scratch_types=[
            pltpu.VMEM((KE, EMB), f32),
            pltpu.VMEM((KE, EMB), f32),
            pltpu.VMEM((KE,), f32),
            pltpu.VMEM((KE,), f32),
            pltpu.VMEM((KE, EMB), f32),
            pltpu.SemaphoreType.DMA,
        ],
    )(emb_ent, agg0, agg1, deg0, deg1)


def _gather_call(seq_t, tid_t, e_embs, time_emb):
    f32 = jnp.float32
    return pl.kernel(
        _gather_body,
        out_type=jax.ShapeDtypeStruct((SEQTOT, EMB), f32),
        mesh=plsc.VectorSubcoreMesh(**_MESH),
        scratch_types=[
            pltpu.VMEM((KS,), jnp.int32),
            pltpu.VMEM((KS,), jnp.int32),
            pltpu.VMEM((KS, EMB), f32),
            pltpu.VMEM((KS, EMB), f32),
            pltpu.SemaphoreType.DMA,
            pltpu.SemaphoreType.DMA,
        ],
    )(seq_t, tid_t, e_embs, time_emb)


def _decode_call(pair3, e_embs, w2, b2):
    nblk = 10
    return pl.pallas_call(
        _decode_body,
        grid=(nblk,),
        in_specs=[
            pl.BlockSpec((SEQ, BATCH, EMB), lambda t: (0, 0, 0)),
            pl.BlockSpec((1024, EMB), lambda t: (t, 0)),
            pl.BlockSpec((1, EMB), lambda t: (0, 0)),
            pl.BlockSpec((1, EMB), lambda t: (0, 0)),
        ],
        out_specs=pl.BlockSpec((BATCH, 1024), lambda t: (0, t)),
        out_shape=jax.ShapeDtypeStruct((BATCH, ENTS), jnp.float32),
        scratch_shapes=[pltpu.VMEM((BATCH, EMB), jnp.float32)],
    )(pair3, e_embs, w2, b2)


def kernel(sequence, time_ids, edge_index, edge_type, emb_ent, emb_rel,
           time_emb, ln_weight, ln_bias):
    i32 = jnp.int32
    src = edge_index[0].astype(i32)
    dst = edge_index[1].astype(i32)
    typ = edge_type.astype(i32) + (
        jnp.arange(NEDGE, dtype=i32) // EPT) * NRELS
    seq_t = sequence.T.reshape(-1).astype(i32)
    tid_t = time_ids.T.reshape(-1).astype(i32)
    rel_rep = jnp.tile(emb_rel, (NW, 1))
    agg0, agg1, deg0, deg1 = _edge_call(src, dst, typ, emb_ent, rel_rep)
    e_embs = _comb_call(emb_ent, agg0, agg1, deg0, deg1)
    pair = _gather_call(seq_t, tid_t, e_embs, time_emb)
    pair3 = pair.reshape(SEQ, BATCH, EMB)
    return _decode_call(pair3, e_embs,
                        ln_weight.reshape(1, EMB).astype(jnp.float32),
                        ln_bias.reshape(1, EMB).astype(jnp.float32))


# speedup vs baseline: 1.0617x; 1.0066x over previous
import math

import jax
import jax.numpy as jnp
from jax import lax
from jax.experimental import pallas as pl
from jax.experimental.pallas import tpu as pltpu
from jax.experimental.pallas import tpu_sc as plsc

EMB = 128
ENTS = 10000
NRELS = 400
NEDGE = 320000
NC = 2
NS = 16
L = 16
NW = NC * NS
EPT = NEDGE // NW
KE = 80
NCHUNK = EPT // KE
CPB = 25
BLK = CPB * KE
NZCH = ENTS // KE
ZIT = -(-NZCH // NS)
CIT = -(-NZCH // NW)
BATCH = 1024
SEQ = 10
SEQTOT = BATCH * SEQ
SPW = SEQTOT // NW
KS = 80
NSC = SPW // KS

_MESH = dict(core_axis_name="c", subcore_axis_name="s")


def _edge_body(src_hbm, dst_hbm, typ_hbm, ent_hbm, rel_hbm,
               agg0, agg1, deg0, deg1, agg_sh, deg_sh,
               src_big, typ_big, dst_v0, dst_v1,
               ent_b0, ent_b1, rel_b0, rel_b1, ones_b, zdeg,
               gsem0, gsem1, ssem0, ssem1):
    c = lax.axis_index("c")
    s = lax.axis_index("s")
    wid = c * NS + s
    ebase = wid * EPT
    zv = jnp.zeros((L,), jnp.float32)
    ov = jnp.ones((L,), jnp.float32)
    ENT = (ent_b0, ent_b1)
    REL = (rel_b0, rel_b1)
    DSTV = (dst_v0, dst_v1)
    GS = (gsem0, gsem1)
    SS = (ssem0, ssem1)

    def _zfill(r, _):
        for jj in range(EMB // L):
            ent_b0[r, pl.ds(jj * L, L)] = zv
        return 0
    lax.fori_loop(0, KE, _zfill, 0)

    def _zfill1(g, _):
        zdeg[pl.ds(g * L, L)] = zv
        ones_b[pl.ds(g * L, L)] = ov
        return 0
    lax.fori_loop(0, KE // L, _zfill1, 0)

    def _zchunk(i, _):
        j = i * NS + s

        @pl.when(j < NZCH)
        def _():
            pltpu.sync_copy(ent_b0, agg_sh.at[pl.ds(j * KE, KE)])
            pltpu.sync_copy(zdeg, deg_sh.at[pl.ds(j * KE, KE)])
        return 0
    lax.fori_loop(0, ZIT, _zchunk, 0)
    plsc.subcore_barrier()

    def load_block(bidx):
        b0 = ebase + bidx * BLK
        pltpu.sync_copy(src_hbm.at[pl.ds(b0, BLK)], src_big)
        pltpu.sync_copy(typ_hbm.at[pl.ds(b0, BLK)], typ_big)

    def issue(cn, q):
        off = (cn % CPB) * KE
        pltpu.async_copy(dst_hbm.at[pl.ds(ebase + cn * KE, KE)], DSTV[q],
                         GS[q])
        pltpu.async_copy(ent_hbm.at[src_big.at[pl.ds(off, KE)]], ENT[q],
                         GS[q])
        pltpu.async_copy(rel_hbm.at[typ_big.at[pl.ds(off, KE)]], REL[q],
                         GS[q])

    def drain_g(p):
        pltpu.make_async_copy(dst_hbm.at[pl.ds(0, KE)], DSTV[p],
                              GS[p]).wait()
        pltpu.make_async_copy(agg0.at[pl.ds(0, KE)], ENT[p], GS[p]).wait()
        pltpu.make_async_copy(agg0.at[pl.ds(0, KE)], REL[p], GS[p]).wait()

    def drain_s(p):
        pltpu.make_async_copy(agg0.at[pl.ds(0, KE)], ENT[p], SS[p]).wait()
        pltpu.make_async_copy(deg0.at[pl.ds(0, KE)], ones_b, SS[p]).wait()

    def mult(p):
        @plsc.parallel_loop(0, KE, unroll=16)
        def _(r):
            for jj in range(EMB // L):
                sl = pl.ds(jj * L, L)
                ENT[p][r, sl] = ENT[p][r, sl] * REL[p][r, sl]

    def scat(p):
        pltpu.async_copy(ENT[p], agg_sh.at[DSTV[p]], SS[p], add=True)
        pltpu.async_copy(ones_b, deg_sh.at[DSTV[p]], SS[p], add=True)

    def step(cn, p, do_next):
        q = 1 - p
        drain_g(p)

        @pl.when(cn >= 1)
        def _():
            drain_s(q)
        if do_next:
            nxt = cn + 1

            @pl.when(nxt % CPB == 0)
            def _():
                load_block(nxt // CPB)
            issue(nxt, q)
        mult(p)
        scat(p)

    load_block(0)
    issue(0, 0)

    def pair(i, _):
        step(2 * i, 0, True)
        step(2 * i + 1, 1, True)
        return 0
    lax.fori_loop(0, (NCHUNK - 1) // 2, pair, 0)
    step(NCHUNK - 1, 0, False)
    drain_s(0)
    plsc.subcore_barrier()

    def _wout(agg_out, deg_out):
        def wchunk(i, _):
            j = i * NS + s

            @pl.when(j < NZCH)
            def _():
                r0 = j * KE
                pltpu.sync_copy(agg_sh.at[pl.ds(r0, KE)], rel_b0)
                pltpu.sync_copy(rel_b0, agg_out.at[pl.ds(r0, KE)])
                pltpu.sync_copy(deg_sh.at[pl.ds(r0, KE)], zdeg)
                pltpu.sync_copy(zdeg, deg_out.at[pl.ds(r0, KE)])
            return 0
        lax.fori_loop(0, ZIT, wchunk, 0)

    @pl.when(c == 0)
    def _():
        _wout(agg0, deg0)

    @pl.when(c == 1)
    def _():
        _wout(agg1, deg1)


def _gather_body(seq_hbm, tid_hbm, ent_hbm, agg0, agg1, deg0, deg1,
                 te_hbm, pair_hbm,
                 sid_v, tid_v, e_b, a0_b, a1_b, t_b, d0_v, d1_v,
                 sem0, sem1):
    c = lax.axis_index("c")
    s = lax.axis_index("s")
    wid = c * NS + s

    def chunk(it, _):
        base = wid * SPW + it * KS
        pltpu.sync_copy(seq_hbm.at[pl.ds(base, KS)], sid_v)
        pltpu.sync_copy(tid_hbm.at[pl.ds(base, KS)], tid_v)
        cps = [
            pltpu.async_copy(ent_hbm.at[sid_v], e_b, sem0),
            pltpu.async_copy(agg0.at[sid_v], a0_b, sem0),
            pltpu.async_copy(agg1.at[sid_v], a1_b, sem0),
            pltpu.async_copy(deg0.at[sid_v], d0_v, sem1),
            pltpu.async_copy(deg1.at[sid_v], d1_v, sem1),
            pltpu.async_copy(te_hbm.at[tid_v], t_b, sem1),
        ]
        for cp in cps:
            cp.wait()

        def grp(g, _):
            dm = jnp.maximum(
                d0_v[pl.ds(g * L, L)] + d1_v[pl.ds(g * L, L)], 1.0)
            for rr in range(L):
                r = g * L + rr
                dv = dm[rr]
                for jj in range(EMB // L):
                    sl = pl.ds(jj * L, L)
                    e_b[r, sl] = e_b[r, sl] + t_b[r, sl] + jnp.maximum(
                        (a0_b[r, sl] + a1_b[r, sl]) / dv, 0.0)
            return 0
        lax.fori_loop(0, KS // L, grp, 0)
        pltpu.sync_copy(e_b, pair_hbm.at[pl.ds(base, KS)])
        return 0
    lax.fori_loop(0, NSC, chunk, 0)


def _decode_body(pair_ref, ent_ref, a0_ref, a1_ref, dg_ref, w_ref, b_ref,
                 out_ref, rep_ref):
    t = pl.program_id(0)

    @pl.when(t == 0)
    def _():
        acc = jnp.zeros((BATCH, EMB), jnp.float32)
        for p in range(SEQ):
            x = pair_ref[p]
            u = jnp.mean(x, axis=-1, keepdims=True)
            xc = x - u
            s2 = jnp.mean(xc * xc, axis=-1, keepdims=True)
            xn = xc / jnp.sqrt(s2 + 1e-12)
            acc = acc + (xn * w_ref[...] + b_ref[...])
        scale = 1.0 / (SEQ * math.sqrt(float(EMB)))
        rep_ref[...] = acc * scale

    e = ent_ref[...] + jnp.maximum(
        (a0_ref[...] + a1_ref[...]) / jnp.maximum(dg_ref[...], 1.0), 0.0)
    out_ref[...] = lax.dot_general(
        rep_ref[...], e, (((1,), (1,)), ((), ())),
        preferred_element_type=jnp.float32,
        precision=lax.Precision.HIGHEST)


def _edge_call(src, dst, typ, emb_ent, emb_rel):
    f32 = jnp.float32
    return pl.kernel(
        _edge_body,
        out_type=[jax.ShapeDtypeStruct((ENTS, EMB), f32),
                  jax.ShapeDtypeStruct((ENTS, EMB), f32),
                  jax.ShapeDtypeStruct((ENTS,), f32),
                  jax.ShapeDtypeStruct((ENTS,), f32)],
        mesh=plsc.VectorSubcoreMesh(**_MESH),
        scratch_types=[
            pltpu.VMEM_SHARED((ENTS, EMB), f32),
            pltpu.VMEM_SHARED((ENTS,), f32),
            pltpu.VMEM((BLK,), jnp.int32),
            pltpu.VMEM((BLK,), jnp.int32),
            pltpu.VMEM((KE,), jnp.int32),
            pltpu.VMEM((KE,), jnp.int32),
            pltpu.VMEM((KE, EMB), f32),
            pltpu.VMEM((KE, EMB), f32),
            pltpu.VMEM((KE, EMB), f32),
            pltpu.VMEM((KE, EMB), f32),
            pltpu.VMEM((KE,), f32),
            pltpu.VMEM((KE,), f32),
            pltpu.SemaphoreType.DMA,
            pltpu.SemaphoreType.DMA,
            pltpu.SemaphoreType.DMA,
            pltpu.SemaphoreType.DMA,
        ],
    )(src, dst, typ, emb_ent, emb_rel)


def _gather_call(seq_t, tid_t, emb_ent, agg0, agg1, deg0, deg1, time_emb):
    f32 = jnp.float32
    return pl.kernel(
        _gather_body,
        out_type=jax.ShapeDtypeStruct((SEQTOT, EMB), f32),
        mesh=plsc.VectorSubcoreMesh(**_MESH),
        scratch_types=[
            pltpu.VMEM((KS,), jnp.int32),
            pltpu.VMEM((KS,), jnp.int32),
            pltpu.VMEM((KS, EMB), f32),
            pltpu.VMEM((KS, EMB), f32),
            pltpu.VMEM((KS, EMB), f32),
            pltpu.VMEM((KS, EMB), f32),
            pltpu.VMEM((KS,), f32),
            pltpu.VMEM((KS,), f32),
            pltpu.SemaphoreType.DMA,
            pltpu.SemaphoreType.DMA,
        ],
    )(seq_t, tid_t, emb_ent, agg0, agg1, deg0, deg1, time_emb)


def _decode_call(pair3, emb_ent, agg0, agg1, dgsum, w2, b2):
    nblk = 10
    return pl.pallas_call(
        _decode_body,
        grid=(nblk,),
        in_specs=[
            pl.BlockSpec((SEQ, BATCH, EMB), lambda t: (0, 0, 0)),
            pl.BlockSpec((1024, EMB), lambda t: (t, 0)),
            pl.BlockSpec((1024, EMB), lambda t: (t, 0)),
            pl.BlockSpec((1024, EMB), lambda t: (t, 0)),
            pl.BlockSpec((1024, 1), lambda t: (t, 0)),
            pl.BlockSpec((1, EMB), lambda t: (0, 0)),
            pl.BlockSpec((1, EMB), lambda t: (0, 0)),
        ],
        out_specs=pl.BlockSpec((BATCH, 1024), lambda t: (0, t)),
        out_shape=jax.ShapeDtypeStruct((BATCH, ENTS), jnp.float32),
        scratch_shapes=[pltpu.VMEM((BATCH, EMB), jnp.float32)],
    )(pair3, emb_ent, agg0, agg1, dgsum, w2, b2)


def kernel(sequence, time_ids, edge_index, edge_type, emb_ent, emb_rel,
           time_emb, ln_weight, ln_bias):
    i32 = jnp.int32
    src = edge_index[0].astype(i32)
    dst = edge_index[1].astype(i32)
    typ = edge_type.astype(i32) + (
        jnp.arange(NEDGE, dtype=i32) // EPT) * NRELS
    seq_t = sequence.T.reshape(-1).astype(i32)
    tid_t = time_ids.T.reshape(-1).astype(i32)
    rel_rep = jnp.tile(emb_rel, (NW, 1))
    agg0, agg1, deg0, deg1 = _edge_call(src, dst, typ, emb_ent, rel_rep)
    pair = _gather_call(seq_t, tid_t, emb_ent, agg0, agg1, deg0, deg1,
                        time_emb)
    pair3 = pair.reshape(SEQ, BATCH, EMB)
    dgsum = (deg0 + deg1).reshape(ENTS, 1)
    return _decode_call(pair3, emb_ent, agg0, agg1, dgsum,
                        ln_weight.reshape(1, EMB).astype(jnp.float32),
                        ln_bias.reshape(1, EMB).astype(jnp.float32))
